# baseline (device time: 58914 ns/iter reference)
import jax
import jax.numpy as jnp
from jax import lax
from jax.experimental import pallas as pl
from jax.experimental.pallas import tpu as pltpu

N_DEV = 16


def kernel(A, B):
    m, _ = A.shape
    _, n = B.shape
    m_per = m // N_DEV

    def body(a_ref, b_ref, out_ref, part_ref, buf_ref, send_sems, recv_sems):
        my = lax.axis_index("i")
        left = lax.rem(my - 1 + N_DEV, N_DEV)
        right = lax.rem(my + 1, N_DEV)

        barrier_sem = pltpu.get_barrier_semaphore()
        for nbr in (left, right):
            pl.semaphore_signal(
                barrier_sem, inc=1,
                device_id=(nbr,), device_id_type=pl.DeviceIdType.MESH,
            )
        pl.semaphore_wait(barrier_sem, 2)

        part_ref[:, :] = jnp.dot(
            a_ref[:, :].astype(jnp.bfloat16),
            b_ref[:, :].astype(jnp.bfloat16),
            preferred_element_type=jnp.float32,
        )

        def chunk(c):
            return part_ref[pl.ds(c * m_per, m_per), :]

        buf_ref[0, :, :] = chunk(lax.rem(my - 1 + N_DEV, N_DEV))

        for s in range(N_DEV - 1):
            rdma = pltpu.make_async_remote_copy(
                src_ref=buf_ref.at[s],
                dst_ref=buf_ref.at[s + 1],
                send_sem=send_sems.at[s],
                recv_sem=recv_sems.at[s],
                device_id=(right,),
                device_id_type=pl.DeviceIdType.MESH,
            )
            rdma.start()
            rdma.wait()
            c = lax.rem(my - 2 - s + 2 * N_DEV, N_DEV)
            if s < N_DEV - 2:
                buf_ref[s + 1, :, :] = buf_ref[s + 1, :, :] + chunk(c)
            else:
                out_ref[:, :] = buf_ref[s + 1, :, :] + chunk(c)

    return pl.pallas_call(
        body,
        out_shape=jax.ShapeDtypeStruct((m_per, n), jnp.float32),
        in_specs=[
            pl.BlockSpec(memory_space=pltpu.VMEM),
            pl.BlockSpec(memory_space=pltpu.VMEM),
        ],
        out_specs=pl.BlockSpec(memory_space=pltpu.VMEM),
        scratch_shapes=[
            pltpu.VMEM((m, n), jnp.float32),
            pltpu.VMEM((N_DEV, m_per, n), jnp.float32),
            pltpu.SemaphoreType.DMA((N_DEV - 1,)),
            pltpu.SemaphoreType.DMA((N_DEV - 1,)),
        ],
        compiler_params=pltpu.CompilerParams(collective_id=0),
    )(A, B)


# device time: 27822 ns/iter; 2.1175x vs baseline; 2.1175x over previous
import jax
import jax.numpy as jnp
from jax import lax
from jax.experimental import pallas as pl
from jax.experimental.pallas import tpu as pltpu

N_DEV = 16
MASKS = (1, 3, 4, 8)


def _span(masks):
    s = {0}
    for m in masks:
        s |= {x ^ m for x in s}
    return sorted(s)


REST = tuple(tuple(_span(MASKS[r + 1:])) for r in range(len(MASKS)))


def kernel(A, B):
    m, _ = A.shape
    _, n = B.shape
    m_per = m // N_DEV

    def body(a_ref, b_ref, out_ref, part_ref,
             s0, s1, s2, s3, r0, r1, r2, r3, send_sems, recv_sems):
        sbufs = (s0, s1, s2, s3)
        rbufs = (r0, r1, r2, r3)
        my = lax.axis_index("i")

        barrier_sem = pltpu.get_barrier_semaphore()
        for mk in MASKS:
            pl.semaphore_signal(
                barrier_sem, inc=1,
                device_id=(my ^ mk,), device_id_type=pl.DeviceIdType.MESH,
            )
        pl.semaphore_wait(barrier_sem, len(MASKS))

        part_ref[:, :] = jnp.dot(
            a_ref[:, :].astype(jnp.bfloat16),
            b_ref[:, :].astype(jnp.bfloat16),
            preferred_element_type=jnp.float32,
        )

        def rows(c):
            return (pl.ds(c * m_per, m_per), slice(None))

        for r, mk in enumerate(MASKS):
            partner = my ^ mk
            for i, e in enumerate(REST[r]):
                sbufs[r][i, :, :] = part_ref[rows(partner ^ e)].astype(
                    jnp.bfloat16)
            rdma = pltpu.make_async_remote_copy(
                src_ref=sbufs[r],
                dst_ref=rbufs[r],
                send_sem=send_sems.at[r],
                recv_sem=recv_sems.at[r],
                device_id=(partner,),
                device_id_type=pl.DeviceIdType.MESH,
            )
            rdma.start()
            rdma.wait()
            for i, e in enumerate(REST[r]):
                part_ref[rows(my ^ e)] = (
                    part_ref[rows(my ^ e)] + rbufs[r][i, :, :].astype(
                        jnp.float32))

        out_ref[:, :] = part_ref[rows(my)]

    cs = [len(REST[r]) for r in range(4)]
    return pl.pallas_call(
        body,
        out_shape=jax.ShapeDtypeStruct((m_per, n), jnp.float32),
        in_specs=[
            pl.BlockSpec(memory_space=pltpu.VMEM),
            pl.BlockSpec(memory_space=pltpu.VMEM),
        ],
        out_specs=pl.BlockSpec(memory_space=pltpu.VMEM),
        scratch_shapes=[
            pltpu.VMEM((m, n), jnp.float32),
            *[pltpu.VMEM((c, m_per, n), jnp.bfloat16) for c in cs],
            *[pltpu.VMEM((c, m_per, n), jnp.bfloat16) for c in cs],
            pltpu.SemaphoreType.DMA((4,)),
            pltpu.SemaphoreType.DMA((4,)),
        ],
        compiler_params=pltpu.CompilerParams(collective_id=0),
    )(A, B)


# device time: 21366 ns/iter; 2.7574x vs baseline; 1.3022x over previous
import jax
import jax.numpy as jnp
from jax import lax
from jax.experimental import pallas as pl
from jax.experimental.pallas import tpu as pltpu

N_DEV = 16


def kernel(A, B):
    m, _ = A.shape
    _, n = B.shape
    m_per = m // N_DEV

    def body(a_ref, b_ref, out_ref, part_ref, sbuf, rbuf,
             send_sems, recv_sems):
        my = lax.axis_index("i")

        barrier_sem = pltpu.get_barrier_semaphore()
        for k in range(1, N_DEV):
            pl.semaphore_signal(
                barrier_sem, inc=1,
                device_id=(lax.rem(my + k, N_DEV),),
                device_id_type=pl.DeviceIdType.MESH,
            )
        pl.semaphore_wait(barrier_sem, N_DEV - 1)

        part_ref[:, :] = jnp.dot(
            a_ref[:, :].astype(jnp.bfloat16),
            b_ref[:, :].astype(jnp.bfloat16),
            preferred_element_type=jnp.float32,
        )

        def rows(c):
            return (pl.ds(c * m_per, m_per), slice(None))

        rdmas = []
        for k in range(N_DEV - 1):
            t = lax.rem(my + 1 + k, N_DEV)
            sbuf[k, :, :] = part_ref[rows(t)].astype(jnp.bfloat16)
            rdma = pltpu.make_async_remote_copy(
                src_ref=sbuf.at[k],
                dst_ref=rbuf.at[k],
                send_sem=send_sems.at[k],
                recv_sem=recv_sems.at[k],
                device_id=(t,),
                device_id_type=pl.DeviceIdType.MESH,
            )
            rdma.start()
            rdmas.append(rdma)

        out_ref[:, :] = part_ref[rows(my)]
        for s in range(N_DEV - 1):
            rdmas[s].wait_recv()
            out_ref[:, :] = out_ref[:, :] + rbuf[s, :, :].astype(jnp.float32)
        for k in range(N_DEV - 1):
            rdmas[k].wait_send()

    return pl.pallas_call(
        body,
        out_shape=jax.ShapeDtypeStruct((m_per, n), jnp.float32),
        in_specs=[
            pl.BlockSpec(memory_space=pltpu.VMEM),
            pl.BlockSpec(memory_space=pltpu.VMEM),
        ],
        out_specs=pl.BlockSpec(memory_space=pltpu.VMEM),
        scratch_shapes=[
            pltpu.VMEM((m, n), jnp.float32),
            pltpu.VMEM((N_DEV - 1, m_per, n), jnp.bfloat16),
            pltpu.VMEM((N_DEV - 1, m_per, n), jnp.bfloat16),
            pltpu.SemaphoreType.DMA((N_DEV - 1,)),
            pltpu.SemaphoreType.DMA((N_DEV - 1,)),
        ],
        compiler_params=pltpu.CompilerParams(collective_id=0),
    )(A, B)


# device time: 20357 ns/iter; 2.8940x vs baseline; 1.0496x over previous
import jax
import jax.numpy as jnp
from jax import lax
from jax.experimental import pallas as pl
from jax.experimental.pallas import tpu as pltpu

N_DEV = 16


def kernel(A, B):
    m, _ = A.shape
    _, n = B.shape
    m_per = m // N_DEV

    def body(a_ref, b_ref, out_ref, part_ref, rbuf, send_sems, recv_sems):
        my = lax.axis_index("i")

        barrier_sem = pltpu.get_barrier_semaphore()
        for k in range(1, N_DEV):
            pl.semaphore_signal(
                barrier_sem, inc=1,
                device_id=(lax.rem(my + k, N_DEV),),
                device_id_type=pl.DeviceIdType.MESH,
            )

        part_ref[:, :] = jnp.dot(
            a_ref[:, :].astype(jnp.bfloat16),
            b_ref[:, :].astype(jnp.bfloat16),
            preferred_element_type=jnp.float32,
        ).astype(jnp.bfloat16)

        pl.semaphore_wait(barrier_sem, N_DEV - 1)

        rdmas = []
        for k in range(N_DEV - 1):
            t = lax.rem(my + 1 + k, N_DEV)
            rdma = pltpu.make_async_remote_copy(
                src_ref=part_ref.at[pl.ds(t * m_per, m_per), :],
                dst_ref=rbuf.at[k],
                send_sem=send_sems.at[k],
                recv_sem=recv_sems.at[k],
                device_id=(t,),
                device_id_type=pl.DeviceIdType.MESH,
            )
            rdma.start()
            rdmas.append(rdma)

        out_ref[:, :] = part_ref[pl.ds(my * m_per, m_per), :].astype(
            jnp.float32)
        for s in range(N_DEV - 1):
            rdmas[s].wait_recv()
            out_ref[:, :] = out_ref[:, :] + rbuf[s, :, :].astype(jnp.float32)
        for k in range(N_DEV - 1):
            rdmas[k].wait_send()

    return pl.pallas_call(
        body,
        out_shape=jax.ShapeDtypeStruct((m_per, n), jnp.float32),
        in_specs=[
            pl.BlockSpec(memory_space=pltpu.VMEM),
            pl.BlockSpec(memory_space=pltpu.VMEM),
        ],
        out_specs=pl.BlockSpec(memory_space=pltpu.VMEM),
        scratch_shapes=[
            pltpu.VMEM((m, n), jnp.bfloat16),
            pltpu.VMEM((N_DEV - 1, m_per, n), jnp.bfloat16),
            pltpu.SemaphoreType.DMA((N_DEV - 1,)),
            pltpu.SemaphoreType.DMA((N_DEV - 1,)),
        ],
        compiler_params=pltpu.CompilerParams(collective_id=0),
    )(A, B)


# device time: 17995 ns/iter; 3.2739x vs baseline; 1.1313x over previous
import jax
import jax.numpy as jnp
from jax import lax
from jax.experimental import pallas as pl
from jax.experimental.pallas import tpu as pltpu

N_DEV = 16
N_PLANE = 4
N_Z = 4


def kernel(A, B):
    m, _ = A.shape
    _, n = B.shape
    m_per = m // N_DEV

    def body(a_ref, b_ref, out_ref, part_ref, cacc, sbuf2, rbuf1, rbuf2,
             ssem1, rsem1, ssem2, rsem2):
        my = lax.axis_index("i")
        q = lax.rem(my, N_PLANE)
        z = my // N_PLANE
        plane_base = my - q

        barrier_sem = pltpu.get_barrier_semaphore()
        for k in range(1, N_PLANE):
            pl.semaphore_signal(
                barrier_sem, inc=1,
                device_id=(plane_base + lax.rem(q + k, N_PLANE),),
                device_id_type=pl.DeviceIdType.MESH,
            )
            pl.semaphore_signal(
                barrier_sem, inc=1,
                device_id=(lax.rem(my + k * N_PLANE, N_DEV),),
                device_id_type=pl.DeviceIdType.MESH,
            )

        part_ref[:, :] = jnp.dot(
            a_ref[:, :].astype(jnp.bfloat16),
            b_ref[:, :].astype(jnp.bfloat16),
            preferred_element_type=jnp.float32,
        ).astype(jnp.bfloat16)

        pl.semaphore_wait(barrier_sem, 2 * (N_PLANE - 1))

        def rows(c):
            return (pl.ds(c * m_per, m_per), slice(None))

        def group_z(h):
            return lax.rem(z + 1 + h, N_Z)

        p1 = {}
        for h in range(N_Z):
            for k in range(N_PLANE - 1):
                t_q = lax.rem(q + 1 + k, N_PLANE)
                rdma = pltpu.make_async_remote_copy(
                    src_ref=part_ref.at[rows(group_z(h) * N_PLANE + t_q)],
                    dst_ref=rbuf1.at[k, h],
                    send_sem=ssem1.at[k, h],
                    recv_sem=rsem1.at[k, h],
                    device_id=(plane_base + t_q,),
                    device_id_type=pl.DeviceIdType.MESH,
                )
                rdma.start()
                p1[k, h] = rdma

        p2 = []
        for h in range(N_Z):
            acc = part_ref[rows(group_z(h) * N_PLANE + q)].astype(jnp.float32)
            for k in range(N_PLANE - 1):
                p1[k, h].wait_recv()
                acc = acc + rbuf1[k, h, :, :].astype(jnp.float32)
            cacc[h, :, :] = acc
            if h < N_Z - 1:
                sbuf2[h, :, :] = acc.astype(jnp.bfloat16)
                rdma = pltpu.make_async_remote_copy(
                    src_ref=sbuf2.at[h],
                    dst_ref=rbuf2.at[h],
                    send_sem=ssem2.at[h],
                    recv_sem=rsem2.at[h],
                    device_id=(group_z(h) * N_PLANE + q,),
                    device_id_type=pl.DeviceIdType.MESH,
                )
                rdma.start()
                p2.append(rdma)

        out_ref[:, :] = cacc[N_Z - 1, :, :]
        for s in range(N_Z - 1):
            p2[s].wait_recv()
            out_ref[:, :] = out_ref[:, :] + rbuf2[s, :, :].astype(jnp.float32)
        for r in p1.values():
            r.wait_send()
        for r in p2:
            r.wait_send()

    return pl.pallas_call(
        body,
        out_shape=jax.ShapeDtypeStruct((m_per, n), jnp.float32),
        in_specs=[
            pl.BlockSpec(memory_space=pltpu.VMEM),
            pl.BlockSpec(memory_space=pltpu.VMEM),
        ],
        out_specs=pl.BlockSpec(memory_space=pltpu.VMEM),
        scratch_shapes=[
            pltpu.VMEM((m, n), jnp.bfloat16),
            pltpu.VMEM((N_Z, m_per, n), jnp.float32),
            pltpu.VMEM((N_Z - 1, m_per, n), jnp.bfloat16),
            pltpu.VMEM((N_PLANE - 1, N_Z, m_per, n), jnp.bfloat16),
            pltpu.VMEM((N_Z - 1, m_per, n), jnp.bfloat16),
            pltpu.SemaphoreType.DMA((N_PLANE - 1, N_Z)),
            pltpu.SemaphoreType.DMA((N_PLANE - 1, N_Z)),
            pltpu.SemaphoreType.DMA((N_Z - 1,)),
            pltpu.SemaphoreType.DMA((N_Z - 1,)),
        ],
        compiler_params=pltpu.CompilerParams(collective_id=0),
    )(A, B)
